# baseline (speedup 1.0000x reference)
import functools

import jax
import jax.numpy as jnp
from jax import lax
from jax.experimental import pallas as pl
from jax.experimental.pallas import tpu as pltpu
from jax.experimental.pallas import tpu_sc as plsc

D_MODEL = 128
BASE_YEAR = 2022
MAX_DELTA = 128
BATCH = 16384

NUM_CORES = 2
NUM_SUBCORES = 16
LANES = 16
NUM_WORKERS = NUM_CORES * NUM_SUBCORES
B_PER_W = BATCH // NUM_WORKERS
CHUNK = 128
N_CHUNKS = B_PER_W // CHUNK


def _make_kernel():
    mesh = plsc.VectorSubcoreMesh(
        core_axis_name="c", subcore_axis_name="s",
        num_cores=NUM_CORES, num_subcores=NUM_SUBCORES,
    )

    @functools.partial(
        pl.kernel,
        mesh=mesh,
        out_type=jax.ShapeDtypeStruct((BATCH, D_MODEL), jnp.float32),
        scratch_types=[
            pltpu.VMEM((B_PER_W,), jnp.int32),
            pltpu.VMEM((N_CHUNKS, CHUNK), jnp.int32),
            pltpu.VMEM((B_PER_W, D_MODEL), jnp.float32),
            pltpu.SemaphoreType.DMA,
        ],
    )
    def k(years_hbm, pe_hbm, out_hbm, yrs_v, idx_v, rows_v, sem):
        wid = lax.axis_index("s") * NUM_CORES + lax.axis_index("c")
        base = wid * B_PER_W
        pltpu.sync_copy(years_hbm.at[pl.ds(base, B_PER_W)], yrs_v)
        per_chunk = CHUNK // LANES
        for i in range(B_PER_W // LANES):
            y = yrs_v[pl.ds(i * LANES, LANES)]
            idx = jnp.clip(y - BASE_YEAR, -MAX_DELTA, MAX_DELTA) + MAX_DELTA
            idx_v[i // per_chunk, pl.ds((i % per_chunk) * LANES, LANES)] = idx
        copies = [
            pltpu.async_copy(
                pe_hbm.at[idx_v.at[j]],
                rows_v.at[pl.ds(j * CHUNK, CHUNK)],
                sem,
            )
            for j in range(N_CHUNKS)
        ]
        for c in copies:
            c.wait()
        pltpu.sync_copy(rows_v, out_hbm.at[pl.ds(base, B_PER_W)])

    return k


_gather = _make_kernel()


@jax.jit
def kernel(years, pe):
    return _gather(years.astype(jnp.int32), pe)


# speedup vs baseline: 3.2049x; 3.2049x over previous
import functools

import jax
import jax.numpy as jnp
from jax import lax
from jax.experimental import pallas as pl
from jax.experimental.pallas import tpu as pltpu
from jax.experimental.pallas import tpu_sc as plsc

D_MODEL = 128
BASE_YEAR = 2022
MAX_DELTA = 128
TABLE_ROWS = 2 * MAX_DELTA + 1
BATCH = 16384

NUM_CORES = 2
NUM_SUBCORES = 16
LANES = 16
NUM_WORKERS = NUM_CORES * NUM_SUBCORES
B_PER_W = BATCH // NUM_WORKERS
N_GROUPS = B_PER_W // LANES


def _make_kernel():
    mesh = plsc.VectorSubcoreMesh(
        core_axis_name="c", subcore_axis_name="s",
        num_cores=NUM_CORES, num_subcores=NUM_SUBCORES,
    )

    @functools.partial(
        pl.kernel,
        mesh=mesh,
        compiler_params=pltpu.CompilerParams(needs_layout_passes=False),
        out_type=jax.ShapeDtypeStruct((BATCH * D_MODEL,), jnp.float32),
        scratch_types=[
            pltpu.VMEM((TABLE_ROWS * D_MODEL,), jnp.float32),
            pltpu.VMEM((B_PER_W,), jnp.int32),
            pltpu.VMEM((B_PER_W * D_MODEL,), jnp.float32),
        ],
    )
    def k(years_hbm, pe_hbm, out_hbm, pe_v, yrs_v, rows_v):
        wid = lax.axis_index("s") * NUM_CORES + lax.axis_index("c")
        base = wid * B_PER_W
        pltpu.sync_copy(pe_hbm, pe_v)
        pltpu.sync_copy(years_hbm.at[pl.ds(base, B_PER_W)], yrs_v)

        lane = lax.iota(jnp.int32, LANES)

        def group(g, carry):
            y = yrs_v[pl.ds(g * LANES, LANES)]
            idx = jnp.clip(y - BASE_YEAR, -MAX_DELTA, MAX_DELTA) + MAX_DELTA
            src = idx * D_MODEL
            dst = (g * LANES + lane) * D_MODEL
            for c in range(D_MODEL):
                v = plsc.load_gather(pe_v, [src + c])
                plsc.store_scatter(rows_v, [dst + c], v)
            return carry

        lax.fori_loop(0, N_GROUPS, group, 0)
        pltpu.sync_copy(rows_v, out_hbm.at[pl.ds(base * D_MODEL, B_PER_W * D_MODEL)])

    return k


_gather = _make_kernel()


@jax.jit
def kernel(years, pe):
    flat = _gather(years.astype(jnp.int32), pe.reshape(-1))
    return flat.reshape(BATCH, D_MODEL)


# speedup vs baseline: 7.5235x; 2.3475x over previous
import functools

import jax
import jax.numpy as jnp
from jax import lax
from jax.experimental import pallas as pl
from jax.experimental.pallas import tpu as pltpu
from jax.experimental.pallas import tpu_sc as plsc

D_MODEL = 128
BASE_YEAR = 2022
MAX_DELTA = 128
TABLE_ROWS = 2 * MAX_DELTA + 1
BATCH = 16384

NUM_CORES = 2
NUM_SUBCORES = 16
LANES = 16
NUM_WORKERS = NUM_CORES * NUM_SUBCORES
B_PER_W = BATCH // NUM_WORKERS
N_GROUPS = B_PER_W // LANES


def _make_kernel():
    mesh = plsc.VectorSubcoreMesh(
        core_axis_name="c", subcore_axis_name="s",
        num_cores=NUM_CORES, num_subcores=NUM_SUBCORES,
    )

    @functools.partial(
        pl.kernel,
        mesh=mesh,
        compiler_params=pltpu.CompilerParams(needs_layout_passes=False),
        out_type=jax.ShapeDtypeStruct((BATCH * D_MODEL,), jnp.float32),
        scratch_types=[
            pltpu.VMEM((TABLE_ROWS * D_MODEL,), jnp.float32),
            pltpu.VMEM((B_PER_W,), jnp.int32),
            pltpu.VMEM((B_PER_W * D_MODEL,), jnp.float32),
        ],
    )
    def k(years_hbm, pe_hbm, out_hbm, pe_v, yrs_v, rows_v):
        wid = lax.axis_index("s") * NUM_CORES + lax.axis_index("c")
        base = wid * B_PER_W
        pltpu.sync_copy(pe_hbm, pe_v)
        pltpu.sync_copy(years_hbm.at[pl.ds(base, B_PER_W)], yrs_v)

        def group(g, carry):
            y = yrs_v[pl.ds(g * LANES, LANES)]
            idx = jnp.clip(y - BASE_YEAR, -MAX_DELTA, MAX_DELTA) + MAX_DELTA
            src = idx * D_MODEL
            for j in range(LANES):
                s = src[j]
                dst = (g * LANES + j) * D_MODEL
                for c in range(0, D_MODEL, LANES):
                    rows_v[pl.ds(dst + c, LANES)] = pe_v[pl.ds(s + c, LANES)]
            return carry

        lax.fori_loop(0, N_GROUPS, group, 0)
        pltpu.sync_copy(rows_v, out_hbm.at[pl.ds(base * D_MODEL, B_PER_W * D_MODEL)])

    return k


_gather = _make_kernel()


@jax.jit
def kernel(years, pe):
    flat = _gather(years.astype(jnp.int32), pe.reshape(-1))
    return flat.reshape(BATCH, D_MODEL)


# speedup vs baseline: 7.8935x; 1.0492x over previous
import functools

import jax
import jax.numpy as jnp
from jax import lax
from jax.experimental import pallas as pl
from jax.experimental.pallas import tpu as pltpu
from jax.experimental.pallas import tpu_sc as plsc

D_MODEL = 128
BASE_YEAR = 2022
MAX_DELTA = 128
TABLE_ROWS = 2 * MAX_DELTA + 1
BATCH = 16384

NUM_CORES = 2
NUM_SUBCORES = 16
LANES = 16
NUM_WORKERS = NUM_CORES * NUM_SUBCORES
B_PER_W = BATCH // NUM_WORKERS
N_GROUPS = B_PER_W // LANES
GROUPS_PER_CHUNK = 4
N_CHUNKS = N_GROUPS // GROUPS_PER_CHUNK
CHUNK_WORDS = GROUPS_PER_CHUNK * LANES * D_MODEL


def _make_kernel():
    mesh = plsc.VectorSubcoreMesh(
        core_axis_name="c", subcore_axis_name="s",
        num_cores=NUM_CORES, num_subcores=NUM_SUBCORES,
    )

    @functools.partial(
        pl.kernel,
        mesh=mesh,
        compiler_params=pltpu.CompilerParams(needs_layout_passes=False),
        out_type=jax.ShapeDtypeStruct((BATCH * D_MODEL,), jnp.float32),
        scratch_types=[
            pltpu.VMEM((TABLE_ROWS * D_MODEL,), jnp.float32),
            pltpu.VMEM((B_PER_W,), jnp.int32),
            pltpu.VMEM((B_PER_W * D_MODEL,), jnp.float32),
            pltpu.SemaphoreType.DMA,
            pltpu.SemaphoreType.DMA,
        ],
    )
    def k(years_hbm, pe_hbm, out_hbm, pe_v, yrs_v, rows_v, in_sem, out_sem):
        wid = lax.axis_index("s") * NUM_CORES + lax.axis_index("c")
        base = wid * B_PER_W
        out_base = base * D_MODEL
        c_pe = pltpu.async_copy(pe_hbm, pe_v, in_sem)
        c_yr = pltpu.async_copy(years_hbm.at[pl.ds(base, B_PER_W)], yrs_v, in_sem)
        c_pe.wait()
        c_yr.wait()

        def chunk(ch, carry):
            for gg in range(GROUPS_PER_CHUNK):
                g = ch * GROUPS_PER_CHUNK + gg
                y = yrs_v[pl.ds(g * LANES, LANES)]
                idx = jnp.clip(y - BASE_YEAR, -MAX_DELTA, MAX_DELTA) + MAX_DELTA
                src = idx * D_MODEL
                for j in range(LANES):
                    s = src[j]
                    dst = (g * LANES + j) * D_MODEL
                    for c in range(0, D_MODEL, LANES):
                        rows_v[pl.ds(dst + c, LANES)] = pe_v[pl.ds(s + c, LANES)]
            pltpu.async_copy(
                rows_v.at[pl.ds(ch * CHUNK_WORDS, CHUNK_WORDS)],
                out_hbm.at[pl.ds(out_base + ch * CHUNK_WORDS, CHUNK_WORDS)],
                out_sem,
            )
            return carry

        lax.fori_loop(0, N_CHUNKS, chunk, 0)
        pltpu.make_async_copy(
            rows_v,
            out_hbm.at[pl.ds(out_base, B_PER_W * D_MODEL)],
            out_sem,
        ).wait()

    return k


_gather = _make_kernel()


@jax.jit
def kernel(years, pe):
    flat = _gather(years.astype(jnp.int32), pe.reshape(-1))
    return flat.reshape(BATCH, D_MODEL)
